# Initial kernel scaffold; baseline (speedup 1.0000x reference)
import jax
import jax.numpy as jnp
from jax.experimental import pallas as pl


def kernel(sequence, padding_idx, W):
    raise NotImplementedError("write your pallas kernel here")


# speedup vs baseline: 267.1133x; 267.1133x over previous
import functools

import jax
import jax.numpy as jnp
from jax import lax
from jax.experimental import pallas as pl
from jax.experimental.pallas import tpu as pltpu
from jax.experimental.pallas import tpu_sc as plsc

NUM_TYPES = 1000
NC = 2
NS = 16
L = 16
NW = NC * NS


def _sc_gather_sum(seq_ext, w_flat, per_tile, chunk):
    nchunk = per_tile // chunk
    mesh = plsc.VectorSubcoreMesh(core_axis_name="c", subcore_axis_name="s")

    @functools.partial(
        pl.kernel,
        out_type=jax.ShapeDtypeStruct((NW, L), jnp.float32),
        mesh=mesh,
        scratch_types=[
            pltpu.VMEM((chunk + L,), jnp.int32),
            pltpu.VMEM((chunk,), jnp.int32),
            pltpu.VMEM((chunk,), jnp.float32),
            pltpu.VMEM((L,), jnp.float32),
            pltpu.SemaphoreType.DMA,
        ],
    )
    def k(seq_hbm, w_hbm, out_hbm, seq_v, idx_v, val_v, acc_v, sem):
        wid = lax.axis_index("s") * NC + lax.axis_index("c")
        base = wid * per_tile

        def chunk_body(ci, acc):
            off = base + ci * chunk
            pltpu.sync_copy(seq_hbm.at[pl.ds(off, chunk + L)], seq_v)

            def build(j, carry):
                a = seq_v[pl.ds(j * L, L)]
                b = seq_v[pl.ds(j * L + 1, L)]
                idx_v[pl.ds(j * L, L)] = a * NUM_TYPES + b
                return carry

            lax.fori_loop(0, chunk // L, build, 0, unroll=4)
            pltpu.async_copy(w_hbm.at[idx_v], val_v, sem).wait()

            def accum(j, a):
                return a + val_v[pl.ds(j * L, L)]

            return lax.fori_loop(0, chunk // L, accum, acc, unroll=4)

        acc = lax.fori_loop(0, nchunk, chunk_body, jnp.zeros((L,), jnp.float32))
        acc_v[...] = acc
        pltpu.sync_copy(acc_v, out_hbm.at[wid])

    return k(seq_ext, w_flat)


def kernel(sequence, padding_idx, W):
    n = sequence.shape[0]
    per_tile = n // NW
    chunk = per_tile
    for c in (25600, 12800, 6400, 3200, 1600, 800, 400, 200, 100):
        if per_tile % c == 0 and 3 * c + 2 * L <= 120000:
            chunk = c
            break
    t = lax.iota(jnp.int32, NUM_TYPES) != padding_idx
    w_flat = jnp.where(t[:, None] & t[None, :], W, 0.0).reshape(-1)
    seq_ext = jnp.concatenate(
        [sequence, jnp.full((L,), padding_idx, dtype=sequence.dtype)])
    parts = _sc_gather_sum(seq_ext, w_flat, per_tile, chunk)
    return -jnp.sum(parts)


# speedup vs baseline: 316.2762x; 1.1841x over previous
import functools

import jax
import jax.numpy as jnp
from jax import lax
from jax.experimental import pallas as pl
from jax.experimental.pallas import tpu as pltpu
from jax.experimental.pallas import tpu_sc as plsc

NUM_TYPES = 1000
NC = 2
NS = 16
L = 16
NW = NC * NS


def _sc_gather_sum(seq_ext, w_flat, per_tile, chunk):
    nchunk = per_tile // chunk
    groups = chunk // L
    mesh = plsc.VectorSubcoreMesh(core_axis_name="c", subcore_axis_name="s")

    @functools.partial(
        pl.kernel,
        out_type=jax.ShapeDtypeStruct((NW, L), jnp.float32),
        mesh=mesh,
        scratch_types=[
            pltpu.VMEM((chunk + L,), jnp.int32),
            pltpu.VMEM((chunk,), jnp.int32),
            pltpu.VMEM((chunk,), jnp.int32),
            pltpu.VMEM((chunk,), jnp.float32),
            pltpu.VMEM((chunk,), jnp.float32),
            pltpu.VMEM((L,), jnp.float32),
            pltpu.SemaphoreType.DMA,
        ],
    )
    def k(seq_hbm, w_hbm, out_hbm, seq_v, idx0, idx1, val0, val1, acc_v, sem):
        wid = lax.axis_index("s") * NC + lax.axis_index("c")
        base = wid * per_tile
        idx_bufs = (idx0, idx1)
        val_bufs = (val0, val1)

        def build(idx_ref):
            def body(j, c):
                a = seq_v[pl.ds(j * L, L)]
                b = seq_v[pl.ds(j * L + 1, L)]
                idx_ref[pl.ds(j * L, L)] = a * NUM_TYPES + b
                return c

            lax.fori_loop(0, groups, body, 0, unroll=4)

        def accum(val_ref, acc):
            def body(j, a):
                return a + val_ref[pl.ds(j * L, L)]

            return lax.fori_loop(0, groups, body, acc, unroll=8)

        pltpu.sync_copy(seq_hbm.at[pl.ds(base, chunk + L)], seq_v)
        build(idx0)
        g = pltpu.async_copy(w_hbm.at[idx0], val0, sem)
        acc = jnp.zeros((L,), jnp.float32)
        for ci in range(1, nchunk):
            cur, nxt = (ci - 1) % 2, ci % 2
            pltpu.sync_copy(
                seq_hbm.at[pl.ds(base + ci * chunk, chunk + L)], seq_v)
            build(idx_bufs[nxt])
            g.wait()
            g = pltpu.async_copy(w_hbm.at[idx_bufs[nxt]], val_bufs[nxt], sem)
            acc = accum(val_bufs[cur], acc)
        g.wait()
        acc = accum(val_bufs[(nchunk - 1) % 2], acc)
        acc_v[...] = acc
        pltpu.sync_copy(acc_v, out_hbm.at[wid])

    return k(seq_ext, w_flat)


def kernel(sequence, padding_idx, W):
    n = sequence.shape[0]
    per_tile = n // NW
    chunk = per_tile
    for c in (20480, 12800, 6400, 3200, 1600, 800, 400, 200, 100):
        if per_tile % c == 0 and 5 * c + 2 * L <= 125000:
            chunk = c
            break
    t = lax.iota(jnp.int32, NUM_TYPES) != padding_idx
    w_flat = jnp.where(t[:, None] & t[None, :], W, 0.0).reshape(-1)
    seq_ext = jnp.concatenate(
        [sequence, jnp.full((L,), padding_idx, dtype=sequence.dtype)])
    parts = _sc_gather_sum(seq_ext, w_flat, per_tile, chunk)
    return -jnp.sum(parts)


# speedup vs baseline: 325.8766x; 1.0304x over previous
import functools

import jax
import jax.numpy as jnp
from jax import lax
from jax.experimental import pallas as pl
from jax.experimental.pallas import tpu as pltpu
from jax.experimental.pallas import tpu_sc as plsc

NUM_TYPES = 1000
NC = 2
NS = 16
L = 16
NW = NC * NS


def _sc_gather_sum(seq, pad16, w_flat, per_tile, chunk):
    nchunk = per_tile // chunk
    groups = chunk // L
    mesh = plsc.VectorSubcoreMesh(core_axis_name="c", subcore_axis_name="s")

    @functools.partial(
        pl.kernel,
        out_type=(jax.ShapeDtypeStruct((NW, L), jnp.float32),
                  jax.ShapeDtypeStruct((NW, L), jnp.int32)),
        mesh=mesh,
        scratch_types=[
            pltpu.VMEM((chunk + L,), jnp.int32),
            pltpu.VMEM((L,), jnp.int32),
            pltpu.VMEM((chunk,), jnp.int32),
            pltpu.VMEM((chunk,), jnp.int32),
            pltpu.VMEM((chunk,), jnp.int32),
            pltpu.VMEM((chunk,), jnp.float32),
            pltpu.VMEM((chunk,), jnp.float32),
            pltpu.VMEM((chunk,), jnp.float32),
            pltpu.VMEM((L,), jnp.float32),
            pltpu.VMEM((L,), jnp.int32),
            pltpu.SemaphoreType.DMA,
            pltpu.SemaphoreType.DMA,
        ],
    )
    def k(seq_hbm, pad_hbm, w_hbm, out_hbm, cnt_hbm,
          seq_v, pad_v, idx0, idx1, idx2, val0, val1, val2,
          acc_v, cac_v, sem0, sem1):
        wid = lax.axis_index("s") * NC + lax.axis_index("c")
        base = wid * per_tile
        idx_bufs = (idx0, idx1, idx2)
        val_bufs = (val0, val1, val2)
        sems = (sem0, sem1)

        pltpu.sync_copy(pad_hbm, pad_v)
        pad = pad_v[...]

        def load_seq(ci):
            off = base + ci * chunk
            if ci == nchunk - 1:
                @pl.when(wid == NW - 1)
                def _():
                    pltpu.sync_copy(seq_hbm.at[pl.ds(off, chunk)],
                                    seq_v.at[pl.ds(0, chunk)])
                    pltpu.sync_copy(pad_hbm, seq_v.at[pl.ds(chunk, L)])

                @pl.when(wid != NW - 1)
                def _():
                    pltpu.sync_copy(seq_hbm.at[pl.ds(off, chunk + L)], seq_v)
            else:
                pltpu.sync_copy(seq_hbm.at[pl.ds(off, chunk + L)], seq_v)

        def build(idx_ref, cnt):
            def body(j, c):
                a = seq_v[pl.ds(j * L, L)]
                b = seq_v[pl.ds(j * L + 1, L)]
                m = (a == pad) | (b == pad)
                f = jnp.where(m, 0, a * NUM_TYPES + b)
                idx_ref[pl.ds(j * L, L)] = f
                return c + jnp.where(m, 1, 0)

            return lax.fori_loop(0, groups, body, cnt, unroll=4)

        def accum(val_ref, acc):
            def body(j, a):
                return a + val_ref[pl.ds(j * L, L)]

            return lax.fori_loop(0, groups, body, acc, unroll=8)

        def start_gather(ci):
            return pltpu.async_copy(
                w_hbm.at[idx_bufs[ci % 3]], val_bufs[ci % 3], sems[ci % 2])

        cnt = jnp.zeros((L,), jnp.int32)
        acc = jnp.zeros((L,), jnp.float32)
        load_seq(0)
        cnt = build(idx0, cnt)
        g0 = start_gather(0)
        load_seq(1)
        cnt = build(idx1, cnt)
        g1 = start_gather(1)
        pending = [g0, g1]
        for ci in range(2, nchunk):
            load_seq(ci)
            cnt = build(idx_bufs[ci % 3], cnt)
            pending[0].wait()
            pending = [pending[1], start_gather(ci)]
            acc = accum(val_bufs[(ci - 2) % 3], acc)
        pending[0].wait()
        acc = accum(val_bufs[(nchunk - 2) % 3], acc)
        pending[1].wait()
        acc = accum(val_bufs[(nchunk - 1) % 3], acc)
        acc_v[...] = acc
        cac_v[...] = cnt
        pltpu.sync_copy(acc_v, out_hbm.at[wid])
        pltpu.sync_copy(cac_v, cnt_hbm.at[wid])

    return k(seq, pad16, w_flat)


def kernel(sequence, padding_idx, W):
    n = sequence.shape[0]
    per_tile = n // NW
    chunk = per_tile
    for c in (12800, 6400, 3200, 1600, 800, 400, 200, 100):
        if per_tile % c == 0 and 7 * c + 4 * L <= 125000:
            chunk = c
            break
    pad16 = jnp.full((L,), padding_idx, dtype=sequence.dtype)
    w_flat = W.reshape(-1)
    parts, cnts = _sc_gather_sum(sequence, pad16, w_flat, per_tile, chunk)
    return -(jnp.sum(parts) - jnp.sum(cnts).astype(jnp.float32) * W[0, 0])


# speedup vs baseline: 564.8369x; 1.7333x over previous
import functools

import jax
import jax.numpy as jnp
from jax import lax
from jax.experimental import pallas as pl
from jax.experimental.pallas import tpu as pltpu
from jax.experimental.pallas import tpu_sc as plsc

NUM_TYPES = 1000
NC = 2
NS = 16
L = 16
NW = NC * NS


def _sc_gather_sum(seq, pad16, w_flat, per_tile, chunk):
    nchunk = per_tile // chunk
    groups = chunk // L
    mesh = plsc.VectorSubcoreMesh(core_axis_name="c", subcore_axis_name="s")

    @functools.partial(
        pl.kernel,
        out_type=(jax.ShapeDtypeStruct((NW, L), jnp.float32),
                  jax.ShapeDtypeStruct((NW, L), jnp.int32)),
        mesh=mesh,
        scratch_types=[
            pltpu.VMEM((chunk + L,), jnp.int32),
            pltpu.VMEM((L,), jnp.int32),
            pltpu.VMEM((chunk,), jnp.int32),
            pltpu.VMEM((chunk,), jnp.int32),
            pltpu.VMEM((chunk,), jnp.int32),
            pltpu.VMEM((chunk,), jnp.float32),
            pltpu.VMEM((chunk,), jnp.float32),
            pltpu.VMEM((chunk,), jnp.float32),
            pltpu.VMEM((L,), jnp.float32),
            pltpu.VMEM((L,), jnp.int32),
            pltpu.VMEM_SHARED((NUM_TYPES * NUM_TYPES,), jnp.float32),
            pltpu.SemaphoreType.DMA,
            pltpu.SemaphoreType.DMA,
        ],
    )
    def k(seq_hbm, pad_hbm, w_hbm, out_hbm, cnt_hbm,
          seq_v, pad_v, idx0, idx1, idx2, val0, val1, val2,
          acc_v, cac_v, w_sh, sem0, sem1):
        wid = lax.axis_index("s") * NC + lax.axis_index("c")
        @pl.when(lax.axis_index("s") == 0)
        def _():
            pltpu.sync_copy(w_hbm, w_sh)

        plsc.subcore_barrier()
        base = wid * per_tile
        idx_bufs = (idx0, idx1, idx2)
        val_bufs = (val0, val1, val2)
        sems = (sem0, sem1)

        pltpu.sync_copy(pad_hbm, pad_v)
        pad = pad_v[...]

        def load_seq(ci):
            off = base + ci * chunk
            if ci == nchunk - 1:
                @pl.when(wid == NW - 1)
                def _():
                    pltpu.sync_copy(seq_hbm.at[pl.ds(off, chunk)],
                                    seq_v.at[pl.ds(0, chunk)])
                    pltpu.sync_copy(pad_hbm, seq_v.at[pl.ds(chunk, L)])

                @pl.when(wid != NW - 1)
                def _():
                    pltpu.sync_copy(seq_hbm.at[pl.ds(off, chunk + L)], seq_v)
            else:
                pltpu.sync_copy(seq_hbm.at[pl.ds(off, chunk + L)], seq_v)

        def build(idx_ref, cnt):
            def body(j, c):
                a = seq_v[pl.ds(j * L, L)]
                b = seq_v[pl.ds(j * L + 1, L)]
                m = (a == pad) | (b == pad)
                f = jnp.where(m, 0, a * NUM_TYPES + b)
                idx_ref[pl.ds(j * L, L)] = f
                return c + jnp.where(m, 1, 0)

            return lax.fori_loop(0, groups, body, cnt, unroll=4)

        def accum(val_ref, acc):
            def body(j, a):
                return a + val_ref[pl.ds(j * L, L)]

            return lax.fori_loop(0, groups, body, acc, unroll=8)

        def start_gather(ci):
            return pltpu.async_copy(
                w_sh.at[idx_bufs[ci % 3]], val_bufs[ci % 3], sems[ci % 2])

        cnt = jnp.zeros((L,), jnp.int32)
        acc = jnp.zeros((L,), jnp.float32)
        load_seq(0)
        cnt = build(idx0, cnt)
        g0 = start_gather(0)
        load_seq(1)
        cnt = build(idx1, cnt)
        g1 = start_gather(1)
        pending = [g0, g1]
        for ci in range(2, nchunk):
            load_seq(ci)
            cnt = build(idx_bufs[ci % 3], cnt)
            pending[0].wait()
            pending = [pending[1], start_gather(ci)]
            acc = accum(val_bufs[(ci - 2) % 3], acc)
        pending[0].wait()
        acc = accum(val_bufs[(nchunk - 2) % 3], acc)
        pending[1].wait()
        acc = accum(val_bufs[(nchunk - 1) % 3], acc)
        acc_v[...] = acc
        cac_v[...] = cnt
        pltpu.sync_copy(acc_v, out_hbm.at[wid])
        pltpu.sync_copy(cac_v, cnt_hbm.at[wid])

    return k(seq, pad16, w_flat)


def kernel(sequence, padding_idx, W):
    n = sequence.shape[0]
    per_tile = n // NW
    chunk = per_tile
    for c in (6400, 3200, 1600, 800, 400, 200, 100):
        if per_tile % c == 0 and 7 * c + 4 * L <= 125000:
            chunk = c
            break
    pad16 = jnp.full((L,), padding_idx, dtype=sequence.dtype)
    w_flat = W.reshape(-1)
    parts, cnts = _sc_gather_sum(sequence, pad16, w_flat, per_tile, chunk)
    return -(jnp.sum(parts) - jnp.sum(cnts).astype(jnp.float32) * W[0, 0])


# speedup vs baseline: 679.0441x; 1.2022x over previous
import functools

import jax
import jax.numpy as jnp
from jax import lax
from jax.experimental import pallas as pl
from jax.experimental.pallas import tpu as pltpu
from jax.experimental.pallas import tpu_sc as plsc

NUM_TYPES = 1000
NC = 2
NS = 16
L = 16
NW = NC * NS
WSZ = NUM_TYPES * NUM_TYPES
WSLICE = 62512


def _sc_gather_sum(seq, pad16, w_flat, per_tile, chunk):
    nchunk = per_tile // chunk
    groups = chunk // L
    mesh = plsc.VectorSubcoreMesh(core_axis_name="c", subcore_axis_name="s")

    @functools.partial(
        pl.kernel,
        out_type=(jax.ShapeDtypeStruct((NW, L), jnp.float32),
                  jax.ShapeDtypeStruct((NW, L), jnp.int32)),
        mesh=mesh,
        scratch_types=[
            pltpu.VMEM((chunk + L,), jnp.int32),
            pltpu.VMEM((chunk + L,), jnp.int32),
            pltpu.VMEM((L,), jnp.int32),
            pltpu.VMEM((chunk,), jnp.int32),
            pltpu.VMEM((chunk,), jnp.int32),
            pltpu.VMEM((chunk,), jnp.int32),
            pltpu.VMEM((chunk,), jnp.float32),
            pltpu.VMEM((chunk,), jnp.float32),
            pltpu.VMEM((chunk,), jnp.float32),
            pltpu.VMEM((L,), jnp.float32),
            pltpu.VMEM((L,), jnp.int32),
            pltpu.VMEM_SHARED((WSZ,), jnp.float32),
            pltpu.SemaphoreType.DMA,
            pltpu.SemaphoreType.DMA,
            pltpu.SemaphoreType.DMA,
            pltpu.SemaphoreType.DMA,
        ],
    )
    def k(seq_hbm, pad_hbm, w_hbm, out_hbm, cnt_hbm,
          seqa, seqb, pad_v, idx0, idx1, idx2, val0, val1, val2,
          acc_v, cac_v, w_sh, sem0, sem1, sseq, swst):
        sid = lax.axis_index("s")
        wid = sid * NC + lax.axis_index("c")
        base = wid * per_tile
        seq_bufs = (seqa, seqb)
        idx_bufs = (idx0, idx1, idx2)
        val_bufs = (val0, val1, val2)
        sems = (sem0, sem1)

        @pl.when(sid == 0)
        def _():
            pltpu.async_copy(w_hbm, w_sh, swst)

        pltpu.sync_copy(pad_hbm, pad_v)
        pad = pad_v[...]

        def start_seq(ci):
            buf = seq_bufs[ci % 2]
            off = base + ci * chunk
            if ci == nchunk - 1:
                @pl.when(wid == NW - 1)
                def _():
                    pltpu.async_copy(seq_hbm.at[pl.ds(off, chunk)],
                                     buf.at[pl.ds(0, chunk)], sseq)
                    pltpu.sync_copy(pad_hbm, buf.at[pl.ds(chunk, L)])

                @pl.when(wid != NW - 1)
                def _():
                    pltpu.async_copy(seq_hbm.at[pl.ds(off, chunk + L)],
                                     buf, sseq)
            else:
                pltpu.async_copy(seq_hbm.at[pl.ds(off, chunk + L)], buf, sseq)

        def wait_seq(ci):
            buf = seq_bufs[ci % 2]
            if ci == nchunk - 1:
                @pl.when(wid == NW - 1)
                def _():
                    pltpu.make_async_copy(seq_hbm.at[pl.ds(0, chunk)],
                                          buf.at[pl.ds(0, chunk)], sseq).wait()

                @pl.when(wid != NW - 1)
                def _():
                    pltpu.make_async_copy(seq_hbm.at[pl.ds(0, chunk + L)],
                                          buf, sseq).wait()
            else:
                pltpu.make_async_copy(seq_hbm.at[pl.ds(0, chunk + L)],
                                      buf, sseq).wait()

        def build(ci, cnt):
            seq_v = seq_bufs[ci % 2]
            idx_ref = idx_bufs[ci % 3]

            def body(j, c):
                a = seq_v[pl.ds(j * L, L)]
                b = seq_v[pl.ds(j * L + 1, L)]
                m = (a == pad) | (b == pad)
                f = jnp.where(m, 0, a * NUM_TYPES + b)
                idx_ref[pl.ds(j * L, L)] = f
                return c + jnp.where(m, 1, 0)

            return lax.fori_loop(0, groups, body, cnt, unroll=4)

        def accum(ci, acc):
            val_ref = val_bufs[ci % 3]

            def body(j, a):
                return a + val_ref[pl.ds(j * L, L)]

            return lax.fori_loop(0, groups, body, acc, unroll=8)

        def start_gather(ci):
            return pltpu.async_copy(
                w_sh.at[idx_bufs[ci % 3]], val_bufs[ci % 3], sems[ci % 2])

        cnt = jnp.zeros((L,), jnp.int32)
        acc = jnp.zeros((L,), jnp.float32)
        start_seq(0)
        pending = [None, None]
        for ci in range(nchunk):
            wait_seq(ci)
            if ci + 1 < nchunk:
                start_seq(ci + 1)
            cnt = build(ci, cnt)
            if ci == 0:
                @pl.when(sid == 0)
                def _():
                    pltpu.make_async_copy(w_hbm, w_sh, swst).wait()

                plsc.subcore_barrier()
            if ci >= 2:
                pending[(ci - 2) % 2].wait()
            new = start_gather(ci)
            if ci >= 2:
                acc = accum(ci - 2, acc)
            pending[ci % 2] = new
        pending[(nchunk - 2) % 2].wait()
        acc = accum(nchunk - 2, acc)
        pending[(nchunk - 1) % 2].wait()
        acc = accum(nchunk - 1, acc)
        acc_v[...] = acc
        cac_v[...] = cnt
        pltpu.sync_copy(acc_v, out_hbm.at[wid])
        pltpu.sync_copy(cac_v, cnt_hbm.at[wid])

    return k(seq, pad16, w_flat)


def kernel(sequence, padding_idx, W):
    n = sequence.shape[0]
    per_tile = n // NW
    chunk = per_tile
    for c in (6400, 3200, 1600, 800, 400, 200, 100):
        if per_tile % c == 0 and 8 * c + 6 * L <= 60000:
            chunk = c
            break
    pad16 = jnp.full((L,), padding_idx, dtype=sequence.dtype)
    w_flat = W.reshape(-1)
    parts, cnts = _sc_gather_sum(sequence, pad16, w_flat, per_tile, chunk)
    return -(jnp.sum(parts) - jnp.sum(cnts).astype(jnp.float32) * W[0, 0])


# speedup vs baseline: 702.7132x; 1.0349x over previous
import functools

import jax
import jax.numpy as jnp
from jax import lax
from jax.experimental import pallas as pl
from jax.experimental.pallas import tpu as pltpu
from jax.experimental.pallas import tpu_sc as plsc

NUM_TYPES = 1000
NC = 2
NS = 16
L = 16
NW = NC * NS
WSZ = NUM_TYPES * NUM_TYPES
NBUF = 4


def _sc_gather_sum(seq, pad16, w_flat, per_tile, chunk):
    nchunk = per_tile // chunk
    groups = chunk // L
    mesh = plsc.VectorSubcoreMesh(core_axis_name="c", subcore_axis_name="s")

    @functools.partial(
        pl.kernel,
        out_type=jax.ShapeDtypeStruct((NW, L), jnp.float32),
        mesh=mesh,
        scratch_types=[
            pltpu.VMEM((chunk + L,), jnp.int32),
            pltpu.VMEM((chunk + L,), jnp.int32),
            pltpu.VMEM((L,), jnp.int32),
            pltpu.VMEM((L,), jnp.float32),
            pltpu.VMEM((chunk,), jnp.int32),
            pltpu.VMEM((chunk,), jnp.int32),
            pltpu.VMEM((chunk,), jnp.int32),
            pltpu.VMEM((chunk,), jnp.int32),
            pltpu.VMEM((chunk,), jnp.float32),
            pltpu.VMEM((chunk,), jnp.float32),
            pltpu.VMEM((chunk,), jnp.float32),
            pltpu.VMEM((chunk,), jnp.float32),
            pltpu.VMEM((L,), jnp.float32),
            pltpu.VMEM_SHARED((WSZ,), jnp.float32),
            pltpu.SemaphoreType.DMA,
            pltpu.SemaphoreType.DMA,
            pltpu.SemaphoreType.DMA,
            pltpu.SemaphoreType.DMA,
            pltpu.SemaphoreType.DMA,
        ],
    )
    def k(seq_hbm, pad_hbm, w_hbm, out_hbm,
          seqa, seqb, pad_v, w0_v, idx0, idx1, idx2, idx3,
          val0, val1, val2, val3, acc_v, w_sh,
          sem0, sem1, sem2, sseq, swst):
        sid = lax.axis_index("s")
        wid = sid * NC + lax.axis_index("c")
        base = wid * per_tile
        seq_bufs = (seqa, seqb)
        idx_bufs = (idx0, idx1, idx2, idx3)
        val_bufs = (val0, val1, val2, val3)
        sems = (sem0, sem1, sem2)

        @pl.when(sid == 0)
        def _():
            pltpu.async_copy(w_hbm, w_sh, swst)

        pltpu.sync_copy(pad_hbm, pad_v)
        pltpu.sync_copy(w_hbm.at[pl.ds(0, L)], w0_v)
        pad = pad_v[...]

        def start_seq(ci):
            buf = seq_bufs[ci % 2]
            off = base + ci * chunk
            if ci == nchunk - 1:
                @pl.when(wid == NW - 1)
                def _():
                    pltpu.async_copy(seq_hbm.at[pl.ds(off, chunk)],
                                     buf.at[pl.ds(0, chunk)], sseq)
                    pltpu.sync_copy(pad_hbm, buf.at[pl.ds(chunk, L)])

                @pl.when(wid != NW - 1)
                def _():
                    pltpu.async_copy(seq_hbm.at[pl.ds(off, chunk + L)],
                                     buf, sseq)
            else:
                pltpu.async_copy(seq_hbm.at[pl.ds(off, chunk + L)], buf, sseq)

        def wait_seq(ci):
            buf = seq_bufs[ci % 2]
            if ci == nchunk - 1:
                @pl.when(wid == NW - 1)
                def _():
                    pltpu.make_async_copy(seq_hbm.at[pl.ds(0, chunk)],
                                          buf.at[pl.ds(0, chunk)], sseq).wait()

                @pl.when(wid != NW - 1)
                def _():
                    pltpu.make_async_copy(seq_hbm.at[pl.ds(0, chunk + L)],
                                          buf, sseq).wait()
            else:
                pltpu.make_async_copy(seq_hbm.at[pl.ds(0, chunk + L)],
                                      buf, sseq).wait()

        def build(ci, cnt):
            seq_v = seq_bufs[ci % 2]
            idx_ref = idx_bufs[ci % NBUF]

            lanes = lax.iota(jnp.int32, L)

            def body(j, c):
                a = seq_v[pl.ds(j * L, L)]
                b = seq_v[pl.ds(j * L + 1, L)]
                m = (a == pad) | (b == pad)
                f = jnp.where(m, lanes, a * NUM_TYPES + b)
                idx_ref[pl.ds(j * L, L)] = f
                return c + jnp.where(m, 1, 0)

            return lax.fori_loop(0, groups, body, cnt, unroll=4)

        def accum(ci, acc):
            val_ref = val_bufs[ci % NBUF]

            def body(j, a):
                return a + val_ref[pl.ds(j * L, L)]

            return lax.fori_loop(0, groups, body, acc, unroll=8)

        def start_gather(ci):
            return pltpu.async_copy(
                w_sh.at[idx_bufs[ci % NBUF]], val_bufs[ci % NBUF],
                sems[ci % 3])

        cnt = jnp.zeros((L,), jnp.int32)
        acc = jnp.zeros((L,), jnp.float32)
        start_seq(0)
        pending = [None, None, None]
        for ci in range(nchunk):
            wait_seq(ci)
            if ci + 1 < nchunk:
                start_seq(ci + 1)
            cnt = build(ci, cnt)
            if ci == 0:
                @pl.when(sid == 0)
                def _():
                    pltpu.make_async_copy(w_hbm, w_sh, swst).wait()

                plsc.subcore_barrier()
            if ci >= 3:
                pending[(ci - 3) % 3].wait()
            new = start_gather(ci)
            if ci >= 3:
                acc = accum(ci - 3, acc)
            pending[ci % 3] = new
        for ci in range(nchunk - 3, nchunk):
            pending[ci % 3].wait()
            acc = accum(ci, acc)
        acc_v[...] = acc - cnt.astype(jnp.float32) * w0_v[...]
        pltpu.sync_copy(acc_v, out_hbm.at[wid])

    return k(seq, pad16, w_flat)


def kernel(sequence, padding_idx, W):
    n = sequence.shape[0]
    per_tile = n // NW
    chunk = per_tile
    for c in (6400, 3200, 1600, 800, 400, 200, 100):
        if per_tile % c == 0 and 10 * c + 8 * L <= 64200:
            chunk = c
            break
    pad16 = jnp.full((L,), padding_idx, dtype=sequence.dtype)
    w_flat = W.reshape(-1)
    parts = _sc_gather_sum(sequence, pad16, w_flat, per_tile, chunk)
    return -jnp.sum(parts)
